# baseline (speedup 1.0000x reference)
import functools
import math

import jax
import jax.numpy as jnp
from jax import lax
from jax.experimental import pallas as pl
from jax.experimental.pallas import tpu as pltpu
from jax.experimental.pallas import tpu_sc as plsc

D = 64
V = 1000000
NTOK = 16384
NJ = 50
SCALE = math.sqrt(D)
LANES = 16

FULL_BLOCKS = V // 128
TAIL_START = FULL_BLOCKS * 128
TAIL_ROWS = V - TAIL_START
NIB = NTOK // 128


def _mesh_info():
    info = plsc.get_sparse_core_info()
    return info.num_cores, info.num_subcores


@functools.lru_cache(maxsize=None)
def _phase1():
    nc, ns = _mesh_info()
    nw = nc * ns
    per_w = FULL_BLOCKS // nw
    extra = FULL_BLOCKS - per_w * nw
    max_b = per_w + 1
    mesh = plsc.VectorSubcoreMesh(core_axis_name="c", subcore_axis_name="s")
    iota = None

    @functools.partial(
        pl.kernel,
        mesh=mesh,
        compiler_params=pltpu.CompilerParams(
            use_tc_tiling_on_sc=True, needs_layout_passes=False),
        out_type=jax.ShapeDtypeStruct((V // 2, 128), jnp.float32),
        scratch_types=(
            [pltpu.VMEM((D, 128), jnp.float32) for _ in range(4)]
            + [pltpu.SemaphoreType.DMA for _ in range(4)]
        ),
    )
    def retile(t2, tail, scr, tin0, tin1, rv0, rv1, r0s, r1s, w0s, w1s):
        tins = (tin0, tin1)
        rvs = (rv0, rv1)
        rsem = (r0s, r1s)
        wsem = (w0s, w1s)
        wid = lax.axis_index("s") * nc + lax.axis_index("c")
        n_b = per_w + jnp.where(wid < extra, 1, 0)
        base = wid * per_w + jnp.minimum(wid, extra)
        iota16 = lax.iota(jnp.int32, LANES)

        def start_read(i, b):
            pltpu.async_copy(
                t2.at[:, pl.ds((base + i) * 128, 128)], tins[b], rsem[b])

        def wait_read(b):
            pltpu.make_async_copy(
                t2.at[:, pl.ds(0, 128)], tins[b], rsem[b]).wait()

        def start_write(i, b):
            pltpu.async_copy(
                rvs[b], scr.at[pl.ds((base + i) * 64, 64)], wsem[b])

        def wait_write(b):
            pltpu.make_async_copy(
                rvs[b], scr.at[pl.ds(0, 64)], wsem[b]).wait()

        def transpose(tin_b, rv_b):
            def row_body(r, _):
                p = r // 2
                c0 = (r % 2) * D
                for d0 in range(0, D, LANES):
                    v = plsc.load_gather(
                        tin_b, [iota16 + d0, jnp.full((LANES,), r, jnp.int32)])
                    rv_b[p, pl.ds(c0 + d0, LANES)] = v
                return 0

            lax.fori_loop(0, 128, row_body, 0, unroll=False)

        start_read(0, 0)

        def outer(o, _):
            for b in range(2):
                i = o * 2 + b

                @pl.when(i < n_b)
                def _():
                    @pl.when(i + 1 < n_b)
                    def _():
                        start_read(i + 1, 1 - b)

                    wait_read(b)

                    @pl.when(i >= 2)
                    def _():
                        wait_write(b)

                    transpose(tins[b], rvs[b])
                    start_write(i, b)
            return 0

        lax.fori_loop(0, (max_b + 1) // 2, outer, 0, unroll=False)
        wait_write(0)
        wait_write(1)

        @pl.when(wid == nw - 1)
        def _():
            pltpu.sync_copy(tail, tins[0])
            transpose(tins[0], rvs[0])
            pltpu.sync_copy(
                rvs[0].at[pl.ds(0, TAIL_ROWS // 2)],
                scr.at[pl.ds(TAIL_START // 2, TAIL_ROWS // 2)])

    return retile


@functools.lru_cache(maxsize=None)
def _phase2():
    nc, ns = _mesh_info()
    nw = nc * ns
    ib_per_w = NIB // nw
    mesh = plsc.VectorSubcoreMesh(core_axis_name="c", subcore_axis_name="s")

    @functools.partial(
        pl.kernel,
        mesh=mesh,
        compiler_params=pltpu.CompilerParams(
            use_tc_tiling_on_sc=False, needs_layout_passes=False),
        out_type=jax.ShapeDtypeStruct((NJ, D // 8, NIB, 8, 128), jnp.float32),
        scratch_types=(
            [pltpu.VMEM((NJ, ib_per_w, 128), jnp.int32)]
            + [pltpu.VMEM((128, D), jnp.float32) for _ in range(4)]
            + [pltpu.VMEM((D // 8, ib_per_w, 8, 128), jnp.float32)
               for _ in range(2)]
            + [pltpu.SemaphoreType.DMA for _ in range(6)]
        ),
    )
    def emb(xt, scr2, out5, idx_v, r0, r1, r2, r3, tv0, tv1,
            g0, g1, g2, g3, s0, s1):
        rows = (r0, r1, r2, r3)
        gsem = (g0, g1, g2, g3)
        tvs = (tv0, tv1)
        ssem = (s0, s1)
        wid = lax.axis_index("s") * nc + lax.axis_index("c")
        w4 = wid * ib_per_w
        iota16 = lax.iota(jnp.int32, LANES)

        pltpu.sync_copy(xt.at[:, pl.ds(w4, ib_per_w)], idx_v)

        def start_gather(q, b):
            pltpu.async_copy(
                scr2.at[idx_v.at[q // ib_per_w, q % ib_per_w]],
                rows[b], gsem[b])

        def wait_gather(b):
            pltpu.make_async_copy(
                scr2.at[idx_v.at[0, 0]], rows[b], gsem[b]).wait()

        def wait_store(par):
            for tr in range(D // 8):
                pltpu.make_async_copy(
                    tvs[par].at[tr], out5.at[0, tr, pl.ds(w4, ib_per_w)],
                    ssem[par]).wait()

        n_chunks = NJ * ib_per_w
        start_gather(0, 0)
        start_gather(1, 1)

        def outer(jo, _):
            for jj in range(2):
                j = jo * 2 + jj
                par = jj

                @pl.when(j >= 2)
                def _():
                    wait_store(par)

                for ibl in range(ib_per_w):
                    q = j * ib_per_w + ibl

                    @pl.when(q + 2 < n_chunks)
                    def _():
                        start_gather(q + 2, (ibl + 2) % 4)

                    wait_gather(ibl)
                    rb = rows[ibl]
                    tvb = tvs[par]

                    def tr_body(r64, _):
                        tr = r64 // 8
                        s = r64 % 8
                        cols = jnp.full((LANES,), r64, jnp.int32)
                        for c0 in range(0, 128, LANES):
                            v = plsc.load_gather(rb, [iota16 + c0, cols])
                            tvb[tr, ibl, s, pl.ds(c0, LANES)] = v * SCALE
                        return 0

                    lax.fori_loop(0, D, tr_body, 0, unroll=False)

                for tr in range(D // 8):
                    pltpu.async_copy(
                        tvs[par].at[tr], out5.at[j, tr, pl.ds(w4, ib_per_w)],
                        ssem[par])
            return 0

        lax.fori_loop(0, NJ // 2, outer, 0, unroll=False)
        wait_store(0)
        wait_store(1)

    return emb


def kernel(x, table):
    xt = x.T.reshape(NJ, NIB, 128).astype(jnp.int32)
    t2 = table.T
    tail = jnp.pad(table[TAIL_START:].T, ((0, 0), (0, 128 - TAIL_ROWS)))
    scr = _phase1()(t2, tail)
    out5 = _phase2()(xt, scr.reshape(V, D))
    return out5.transpose(2, 4, 0, 1, 3).reshape(NTOK, NJ, D)


# speedup vs baseline: 1.2085x; 1.2085x over previous
import functools
import math

import jax
import jax.numpy as jnp
from jax import lax
from jax.experimental import pallas as pl
from jax.experimental.pallas import tpu as pltpu
from jax.experimental.pallas import tpu_sc as plsc

D = 64
V = 1000000
NTOK = 16384
NJ = 50
SCALE = math.sqrt(D)
LANES = 16

FULL_BLOCKS = V // 128
TAIL_START = FULL_BLOCKS * 128
TAIL_ROWS = V - TAIL_START
NIB = NTOK // 128


def _mesh_info():
    info = plsc.get_sparse_core_info()
    return info.num_cores, info.num_subcores


@functools.lru_cache(maxsize=None)
def _phase1():
    nc, ns = _mesh_info()
    nw = nc * ns
    per_w = FULL_BLOCKS // nw
    extra = FULL_BLOCKS - per_w * nw
    max_b = per_w + 1
    mesh = plsc.VectorSubcoreMesh(core_axis_name="c", subcore_axis_name="s")
    iota = None

    @functools.partial(
        pl.kernel,
        mesh=mesh,
        compiler_params=pltpu.CompilerParams(
            use_tc_tiling_on_sc=True, needs_layout_passes=False),
        out_type=jax.ShapeDtypeStruct((V // 2, 128), jnp.float32),
        scratch_types=(
            [pltpu.VMEM((D, 128), jnp.float32) for _ in range(4)]
            + [pltpu.SemaphoreType.DMA for _ in range(4)]
        ),
    )
    def retile(t2, tail, scr, tin0, tin1, rv0, rv1, r0s, r1s, w0s, w1s):
        tins = (tin0, tin1)
        rvs = (rv0, rv1)
        rsem = (r0s, r1s)
        wsem = (w0s, w1s)
        wid = lax.axis_index("s") * nc + lax.axis_index("c")
        n_b = per_w + jnp.where(wid < extra, 1, 0)
        base = wid * per_w + jnp.minimum(wid, extra)
        iota16 = lax.iota(jnp.int32, LANES)

        def start_read(i, b):
            pltpu.async_copy(
                t2.at[:, pl.ds((base + i) * 128, 128)], tins[b], rsem[b])

        def wait_read(b):
            pltpu.make_async_copy(
                t2.at[:, pl.ds(0, 128)], tins[b], rsem[b]).wait()

        def start_write(i, b):
            pltpu.async_copy(
                rvs[b], scr.at[pl.ds((base + i) * 64, 64)], wsem[b])

        def wait_write(b):
            pltpu.make_async_copy(
                rvs[b], scr.at[pl.ds(0, 64)], wsem[b]).wait()

        iota64 = iota16 * D

        def transpose(tin_b, rv_b):
            def d_body(d, _):
                for c0 in range(0, 128, LANES):
                    flat = iota64 + (c0 * D + d)
                    v = tin_b[d, pl.ds(c0, LANES)]
                    plsc.store_scatter(rv_b, [flat >> 7, flat & 127], v)
                return 0

            lax.fori_loop(0, D, d_body, 0, unroll=False)

        start_read(0, 0)

        def outer(o, _):
            for b in range(2):
                i = o * 2 + b

                @pl.when(i < n_b)
                def _():
                    @pl.when(i + 1 < n_b)
                    def _():
                        start_read(i + 1, 1 - b)

                    wait_read(b)

                    @pl.when(i >= 2)
                    def _():
                        wait_write(b)

                    transpose(tins[b], rvs[b])
                    start_write(i, b)
            return 0

        lax.fori_loop(0, (max_b + 1) // 2, outer, 0, unroll=False)
        wait_write(0)
        wait_write(1)

        @pl.when(wid == nw - 1)
        def _():
            pltpu.sync_copy(tail, tins[0])
            transpose(tins[0], rvs[0])
            pltpu.sync_copy(
                rvs[0].at[pl.ds(0, TAIL_ROWS // 2)],
                scr.at[pl.ds(TAIL_START // 2, TAIL_ROWS // 2)])

    return retile


@functools.lru_cache(maxsize=None)
def _phase2():
    nc, ns = _mesh_info()
    nw = nc * ns
    ib_per_w = NIB // nw
    mesh = plsc.VectorSubcoreMesh(core_axis_name="c", subcore_axis_name="s")

    @functools.partial(
        pl.kernel,
        mesh=mesh,
        compiler_params=pltpu.CompilerParams(
            use_tc_tiling_on_sc=False, needs_layout_passes=False),
        out_type=jax.ShapeDtypeStruct((NJ, D // 8, NIB, 8, 128), jnp.float32),
        scratch_types=(
            [pltpu.VMEM((NJ, ib_per_w, 128), jnp.int32)]
            + [pltpu.VMEM((128, D), jnp.float32) for _ in range(4)]
            + [pltpu.VMEM((D // 8, ib_per_w, 8, 128), jnp.float32)
               for _ in range(2)]
            + [pltpu.SemaphoreType.DMA for _ in range(6)]
        ),
    )
    def emb(xt, scr2, out5, idx_v, r0, r1, r2, r3, tv0, tv1,
            g0, g1, g2, g3, s0, s1):
        rows = (r0, r1, r2, r3)
        gsem = (g0, g1, g2, g3)
        tvs = (tv0, tv1)
        ssem = (s0, s1)
        wid = lax.axis_index("s") * nc + lax.axis_index("c")
        w4 = wid * ib_per_w
        iota16 = lax.iota(jnp.int32, LANES)

        pltpu.sync_copy(xt.at[:, pl.ds(w4, ib_per_w)], idx_v)

        def start_gather(q, b):
            pltpu.async_copy(
                scr2.at[idx_v.at[q // ib_per_w, q % ib_per_w]],
                rows[b], gsem[b])

        def wait_gather(b):
            pltpu.make_async_copy(
                scr2.at[idx_v.at[0, 0]], rows[b], gsem[b]).wait()

        def wait_store(par):
            for tr in range(D // 8):
                pltpu.make_async_copy(
                    tvs[par].at[tr], out5.at[0, tr, pl.ds(w4, ib_per_w)],
                    ssem[par]).wait()

        n_chunks = NJ * ib_per_w
        start_gather(0, 0)
        start_gather(1, 1)

        def outer(jo, _):
            for jj in range(2):
                j = jo * 2 + jj
                par = jj

                @pl.when(j >= 2)
                def _():
                    wait_store(par)

                for ibl in range(ib_per_w):
                    q = j * ib_per_w + ibl

                    @pl.when(q + 2 < n_chunks)
                    def _():
                        start_gather(q + 2, (ibl + 2) % 4)

                    wait_gather(ibl)
                    rb = rows[ibl]
                    tvb = tvs[par]
                    ibl16 = jnp.full((LANES,), ibl, jnp.int32)

                    def tok_body(r, _):
                        r16 = jnp.full((LANES,), r, jnp.int32)
                        for d0 in range(0, D, LANES):
                            d16 = iota16 + d0
                            v = rb[r, pl.ds(d0, LANES)] * SCALE
                            plsc.store_scatter(
                                tvb, [d16 >> 3, ibl16, d16 & 7, r16], v)
                        return 0

                    lax.fori_loop(0, 128, tok_body, 0, unroll=False)

                for tr in range(D // 8):
                    pltpu.async_copy(
                        tvs[par].at[tr], out5.at[j, tr, pl.ds(w4, ib_per_w)],
                        ssem[par])
            return 0

        lax.fori_loop(0, NJ // 2, outer, 0, unroll=False)
        wait_store(0)
        wait_store(1)

    return emb


def kernel(x, table):
    xt = x.T.reshape(NJ, NIB, 128).astype(jnp.int32)
    t2 = table.T
    tail = jnp.pad(table[TAIL_START:].T, ((0, 0), (0, 128 - TAIL_ROWS)))
    scr = _phase1()(t2, tail)
    out5 = _phase2()(xt, scr.reshape(V, D))
    return out5.transpose(2, 4, 0, 1, 3).reshape(NTOK, NJ, D)


# speedup vs baseline: 1.2152x; 1.0055x over previous
import functools
import math

import jax
import jax.numpy as jnp
import numpy as np
from jax import lax
from jax.experimental import pallas as pl
from jax.experimental.pallas import tpu as pltpu
from jax.experimental.pallas import tpu_sc as plsc

D = 64
V = 1000000
NTOK = 16384
NJ = 50
SCALE = math.sqrt(D)
LANES = 16

FULL_BLOCKS = V // 128
TAIL_START = FULL_BLOCKS * 128
TAIL_ROWS = V - TAIL_START
NIB = NTOK // 128


def _mesh_info():
    info = plsc.get_sparse_core_info()
    return info.num_cores, info.num_subcores


@functools.lru_cache(maxsize=None)
def _phase1():
    nc, ns = _mesh_info()
    nw = nc * ns
    per_w = FULL_BLOCKS // nw
    extra = FULL_BLOCKS - per_w * nw
    max_b = per_w + 1
    mesh = plsc.VectorSubcoreMesh(core_axis_name="c", subcore_axis_name="s")
    iota = None

    @functools.partial(
        pl.kernel,
        mesh=mesh,
        compiler_params=pltpu.CompilerParams(
            use_tc_tiling_on_sc=True, needs_layout_passes=False),
        out_type=jax.ShapeDtypeStruct((V * D,), jnp.float32),
        scratch_types=(
            [pltpu.VMEM((D, 128), jnp.float32) for _ in range(2)]
            + [pltpu.VMEM((128 * D,), jnp.float32) for _ in range(2)]
            + [pltpu.SemaphoreType.DMA for _ in range(4)]
        ),
    )
    def retile(t2, tail, scr, tin0, tin1, rv0, rv1, r0s, r1s, w0s, w1s):
        tins = (tin0, tin1)
        rvs = (rv0, rv1)
        rsem = (r0s, r1s)
        wsem = (w0s, w1s)
        wid = lax.axis_index("s") * nc + lax.axis_index("c")
        n_b = per_w + jnp.where(wid < extra, 1, 0)
        base = wid * per_w + jnp.minimum(wid, extra)
        iota16 = lax.iota(jnp.int32, LANES)
        consts = [(iota16 + c0) * D for c0 in range(0, 128, LANES)]

        def start_read(i, b):
            pltpu.async_copy(
                t2.at[:, pl.ds((base + i) * 128, 128)], tins[b], rsem[b])

        def wait_read(b):
            pltpu.make_async_copy(
                t2.at[:, pl.ds(0, 128)], tins[b], rsem[b]).wait()

        def start_write(i, b):
            pltpu.async_copy(
                rvs[b], scr.at[pl.ds((base + i) * 128 * D, 128 * D)], wsem[b])

        def wait_write(b):
            pltpu.make_async_copy(
                rvs[b], scr.at[pl.ds(0, 128 * D)], wsem[b]).wait()

        def transpose(tin_b, rv_b):
            def d_body(i, _):
                for u in range(4):
                    d = i * 4 + u
                    for c0 in range(8):
                        v = tin_b[d, pl.ds(c0 * LANES, LANES)]
                        plsc.store_scatter(rv_b, [consts[c0] + d], v)
                return 0

            lax.fori_loop(0, D // 4, d_body, 0, unroll=False)

        start_read(0, 0)

        def outer(o, _):
            for b in range(2):
                i = o * 2 + b

                @pl.when(i < n_b)
                def _():
                    @pl.when(i + 1 < n_b)
                    def _():
                        start_read(i + 1, 1 - b)

                    wait_read(b)

                    @pl.when(i >= 2)
                    def _():
                        wait_write(b)

                    transpose(tins[b], rvs[b])
                    start_write(i, b)
            return 0

        lax.fori_loop(0, (max_b + 1) // 2, outer, 0, unroll=False)
        wait_write(0)
        wait_write(1)

        @pl.when(wid == nw - 1)
        def _():
            pltpu.sync_copy(tail, tins[0])
            transpose(tins[0], rvs[0])
            pltpu.sync_copy(
                rvs[0].at[pl.ds(0, TAIL_ROWS * D)],
                scr.at[pl.ds(TAIL_START * D, TAIL_ROWS * D)])

    return retile


@functools.lru_cache(maxsize=None)
def _phase2():
    nc, ns = _mesh_info()
    nw = nc * ns
    ib_per_w = NIB // nw
    mesh = plsc.VectorSubcoreMesh(core_axis_name="c", subcore_axis_name="s")

    @functools.partial(
        pl.kernel,
        mesh=mesh,
        compiler_params=pltpu.CompilerParams(
            use_tc_tiling_on_sc=False, needs_layout_passes=False),
        out_type=jax.ShapeDtypeStruct((NJ, D // 8, NIB, 8, 128), jnp.float32),
        scratch_types=(
            [pltpu.VMEM((NJ, ib_per_w, 128), jnp.int32)]
            + [pltpu.VMEM((128, D), jnp.float32) for _ in range(4)]
            + [pltpu.VMEM((D // 8, ib_per_w, 8, 128), jnp.float32)
               for _ in range(2)]
            + [pltpu.SemaphoreType.DMA for _ in range(6)]
        ),
    )
    def emb(xt, scr2, out5, idx_v, r0, r1, r2, r3, tv0, tv1,
            g0, g1, g2, g3, s0, s1):
        rows = (r0, r1, r2, r3)
        gsem = (g0, g1, g2, g3)
        tvs = (tv0, tv1)
        ssem = (s0, s1)
        wid = lax.axis_index("s") * nc + lax.axis_index("c")
        w4 = wid * ib_per_w
        iota16 = lax.iota(jnp.int32, LANES)
        tr_vec = [(iota16 + d0) >> 3 for d0 in range(0, D, LANES)]
        s_vec = [(iota16 + d0) & 7 for d0 in range(0, D, LANES)]
        ibl_vec = [jnp.full((LANES,), ibl, jnp.int32)
                   for ibl in range(ib_per_w)]

        pltpu.sync_copy(xt.at[:, pl.ds(w4, ib_per_w)], idx_v)

        def start_gather(q, b):
            pltpu.async_copy(
                scr2.at[idx_v.at[q // ib_per_w, q % ib_per_w]],
                rows[b], gsem[b])

        def wait_gather(b):
            pltpu.make_async_copy(
                scr2.at[idx_v.at[0, 0]], rows[b], gsem[b]).wait()

        def store_slab(j, par, tr):
            return pltpu.async_copy(
                tvs[par].at[tr], out5.at[j, tr, pl.ds(w4, ib_per_w)],
                ssem[par])

        def wait_store(par):
            for tr in range(D // 8):
                pltpu.make_async_copy(
                    tvs[par].at[tr], out5.at[0, tr, pl.ds(w4, ib_per_w)],
                    ssem[par]).wait()

        n_chunks = NJ * ib_per_w
        start_gather(0, 0)
        start_gather(1, 1)

        def outer(jo, _):
            for jj in range(2):
                j = jo * 2 + jj
                par = jj

                @pl.when(j >= 2)
                def _():
                    wait_store(par)

                for ibl in range(ib_per_w):
                    q = j * ib_per_w + ibl

                    @pl.when(q + 2 < n_chunks)
                    def _():
                        start_gather(q + 2, (ibl + 2) % 4)

                    wait_gather(ibl)
                    rb = rows[ibl]
                    tvb = tvs[par]
                    iblv = ibl_vec[ibl]

                    def tok_body(i, _):
                        for u in range(4):
                            r = i * 4 + u
                            r16 = jnp.full((LANES,), r, jnp.int32)
                            for g, d0 in enumerate(range(0, D, LANES)):
                                v = rb[r, pl.ds(d0, LANES)] * SCALE
                                plsc.store_scatter(
                                    tvb, [tr_vec[g], iblv, s_vec[g], r16], v)
                        return 0

                    lax.fori_loop(0, 32, tok_body, 0, unroll=False)

                for tr in range(D // 8):
                    store_slab(j, par, tr)
            return 0

        lax.fori_loop(0, NJ // 2, outer, 0, unroll=False)
        wait_store(0)
        wait_store(1)

    return emb


def kernel(x, table):
    xt = x.T.reshape(NJ, NIB, 128).astype(jnp.int32)
    t2 = table.T
    tail = jnp.pad(table[TAIL_START:].T, ((0, 0), (0, 128 - TAIL_ROWS)))
    scr = _phase1()(t2, tail)
    out5 = _phase2()(xt, scr.reshape(V, D))
    return out5.transpose(2, 4, 0, 1, 3).reshape(NTOK, NJ, D)


# speedup vs baseline: 4.1366x; 3.4041x over previous
import functools
import math

import jax
import jax.numpy as jnp
import numpy as np
from jax import lax
from jax.experimental import pallas as pl
from jax.experimental.pallas import tpu as pltpu
from jax.experimental.pallas import tpu_sc as plsc

D = 64
V = 1000000
NTOK = 16384
NJ = 50
SCALE = math.sqrt(D)
LANES = 16

FULL_BLOCKS = V // 128
TAIL_START = FULL_BLOCKS * 128
TAIL_ROWS = V - TAIL_START
NIB = NTOK // 128


def _mesh_info():
    info = plsc.get_sparse_core_info()
    return info.num_cores, info.num_subcores


@functools.lru_cache(maxsize=None)
def _phase1():
    nc, ns = _mesh_info()
    nw = nc * ns
    per_w = FULL_BLOCKS // nw
    extra = FULL_BLOCKS - per_w * nw
    max_b = per_w + 1
    mesh = plsc.VectorSubcoreMesh(core_axis_name="c", subcore_axis_name="s")
    iota = None

    @functools.partial(
        pl.kernel,
        mesh=mesh,
        compiler_params=pltpu.CompilerParams(
            use_tc_tiling_on_sc=True, needs_layout_passes=False),
        out_type=jax.ShapeDtypeStruct((V * D,), jnp.float32),
        scratch_types=(
            [pltpu.VMEM((D, 128), jnp.float32) for _ in range(2)]
            + [pltpu.VMEM((128 * D,), jnp.float32) for _ in range(2)]
            + [pltpu.SemaphoreType.DMA for _ in range(4)]
        ),
    )
    def retile(t2, tail, scr, tin0, tin1, rv0, rv1, r0s, r1s, w0s, w1s):
        tins = (tin0, tin1)
        rvs = (rv0, rv1)
        rsem = (r0s, r1s)
        wsem = (w0s, w1s)
        wid = lax.axis_index("s") * nc + lax.axis_index("c")
        n_b = per_w + jnp.where(wid < extra, 1, 0)
        base = wid * per_w + jnp.minimum(wid, extra)
        iota16 = lax.iota(jnp.int32, LANES)
        perm = [(iota16 + k) & 15 for k in range(LANES)]
        ivst = [iota16 * D + perm[k] for k in range(LANES)]

        def start_read(i, b):
            pltpu.async_copy(
                t2.at[:, pl.ds((base + i) * 128, 128)], tins[b], rsem[b])

        def wait_read(b):
            pltpu.make_async_copy(
                t2.at[:, pl.ds(0, 128)], tins[b], rsem[b]).wait()

        def start_write(i, b):
            pltpu.async_copy(
                rvs[b], scr.at[pl.ds((base + i) * 128 * D, 128 * D)], wsem[b])

        def wait_write(b):
            pltpu.make_async_copy(
                rvs[b], scr.at[pl.ds(0, 128 * D)], wsem[b]).wait()

        def transpose(tin_b, rv_b):
            def blk_body(t, _):
                d0 = (t // 8) * LANES
                c0 = (t % 8) * LANES
                cols = iota16 + c0
                off = c0 * D + d0
                vs = [plsc.load_gather(tin_b, [perm[k] + d0, cols])
                      for k in range(LANES)]
                for k in range(LANES):
                    plsc.store_scatter(rv_b, [ivst[k] + off], vs[k])
                return 0

            lax.fori_loop(0, (D // LANES) * 8, blk_body, 0, unroll=False)

        start_read(0, 0)

        def outer(o, _):
            for b in range(2):
                i = o * 2 + b

                @pl.when(i < n_b)
                def _():
                    @pl.when(i + 1 < n_b)
                    def _():
                        start_read(i + 1, 1 - b)

                    wait_read(b)

                    @pl.when(i >= 2)
                    def _():
                        wait_write(b)

                    transpose(tins[b], rvs[b])
                    start_write(i, b)
            return 0

        lax.fori_loop(0, (max_b + 1) // 2, outer, 0, unroll=False)
        wait_write(0)
        wait_write(1)

        @pl.when(wid == nw - 1)
        def _():
            pltpu.sync_copy(tail, tins[0])
            transpose(tins[0], rvs[0])
            pltpu.sync_copy(
                rvs[0].at[pl.ds(0, TAIL_ROWS * D)],
                scr.at[pl.ds(TAIL_START * D, TAIL_ROWS * D)])

    return retile


@functools.lru_cache(maxsize=None)
def _phase2():
    nc, ns = _mesh_info()
    nw = nc * ns
    ib_per_w = NIB // nw
    mesh = plsc.VectorSubcoreMesh(core_axis_name="c", subcore_axis_name="s")

    @functools.partial(
        pl.kernel,
        mesh=mesh,
        compiler_params=pltpu.CompilerParams(
            use_tc_tiling_on_sc=False, needs_layout_passes=False),
        out_type=jax.ShapeDtypeStruct((NJ, D // 8, NIB, 8, 128), jnp.float32),
        scratch_types=(
            [pltpu.VMEM((NJ, ib_per_w, 128), jnp.int32)]
            + [pltpu.VMEM((128, D), jnp.float32) for _ in range(4)]
            + [pltpu.VMEM((D // 8, ib_per_w, 8, 128), jnp.float32)
               for _ in range(2)]
            + [pltpu.SemaphoreType.DMA for _ in range(6)]
        ),
    )
    def emb(xt, scr2, out5, idx_v, r0, r1, r2, r3, tv0, tv1,
            g0, g1, g2, g3, s0, s1):
        rows = (r0, r1, r2, r3)
        gsem = (g0, g1, g2, g3)
        tvs = (tv0, tv1)
        ssem = (s0, s1)
        wid = lax.axis_index("s") * nc + lax.axis_index("c")
        w4 = wid * ib_per_w
        iota16 = lax.iota(jnp.int32, LANES)
        perm = [(iota16 + k) & 15 for k in range(LANES)]
        permhi = [perm[k] >> 3 for k in range(LANES)]
        permlo = [perm[k] & 7 for k in range(LANES)]
        ibl_vec = [jnp.full((LANES,), ibl, jnp.int32)
                   for ibl in range(ib_per_w)]

        pltpu.sync_copy(xt.at[:, pl.ds(w4, ib_per_w)], idx_v)

        def start_gather(q, b):
            pltpu.async_copy(
                scr2.at[idx_v.at[q // ib_per_w, q % ib_per_w]],
                rows[b], gsem[b])

        def wait_gather(b):
            pltpu.make_async_copy(
                scr2.at[idx_v.at[0, 0]], rows[b], gsem[b]).wait()

        def store_slab(j, par, tr):
            return pltpu.async_copy(
                tvs[par].at[tr], out5.at[j, tr, pl.ds(w4, ib_per_w)],
                ssem[par])

        def wait_store(par):
            for tr in range(D // 8):
                pltpu.make_async_copy(
                    tvs[par].at[tr], out5.at[0, tr, pl.ds(w4, ib_per_w)],
                    ssem[par]).wait()

        n_chunks = NJ * ib_per_w
        start_gather(0, 0)
        start_gather(1, 1)

        def outer(jo, _):
            for jj in range(2):
                j = jo * 2 + jj
                par = jj

                @pl.when(j >= 2)
                def _():
                    wait_store(par)

                for ibl in range(ib_per_w):
                    q = j * ib_per_w + ibl

                    @pl.when(q + 2 < n_chunks)
                    def _():
                        start_gather(q + 2, (ibl + 2) % 4)

                    wait_gather(ibl)
                    rb = rows[ibl]
                    tvb = tvs[par]
                    iblv = ibl_vec[ibl]

                    def blk_body(t, _):
                        d0 = (t // 8) * LANES
                        c0 = (t % 8) * LANES
                        trb = (t // 8) * 2
                        cols = iota16 + c0
                        vs = [plsc.load_gather(rb, [cols, perm[k] + d0])
                              for k in range(LANES)]
                        for k in range(LANES):
                            plsc.store_scatter(
                                tvb,
                                [permhi[k] + trb, iblv, permlo[k], cols],
                                vs[k] * SCALE)
                        return 0

                    lax.fori_loop(0, (D // LANES) * 8, blk_body, 0,
                                  unroll=False)

                for tr in range(D // 8):
                    store_slab(j, par, tr)
            return 0

        lax.fori_loop(0, NJ // 2, outer, 0, unroll=False)
        wait_store(0)
        wait_store(1)

    return emb


def kernel(x, table):
    xt = x.T.reshape(NJ, NIB, 128).astype(jnp.int32)
    t2 = table.T
    tail = jnp.pad(table[TAIL_START:].T, ((0, 0), (0, 128 - TAIL_ROWS)))
    scr = _phase1()(t2, tail)
    out5 = _phase2()(xt, scr.reshape(V, D))
    return out5.transpose(2, 4, 0, 1, 3).reshape(NTOK, NJ, D)


# speedup vs baseline: 4.1436x; 1.0017x over previous
import functools
import math

import jax
import jax.numpy as jnp
from jax import lax
from jax.experimental import pallas as pl
from jax.experimental.pallas import tpu as pltpu
from jax.experimental.pallas import tpu_sc as plsc

D = 64
V = 1000000
NTOK = 16384
NJ = 50
SCALE = math.sqrt(D)
LANES = 16

FULL_BLOCKS = V // 128
TAIL_START = FULL_BLOCKS * 128
TAIL_ROWS = V - TAIL_START
NIB = NTOK // 128


def _mesh_info():
    info = plsc.get_sparse_core_info()
    return info.num_cores, info.num_subcores


@functools.lru_cache(maxsize=None)
def _phase1():
    nc, ns = _mesh_info()
    nw = nc * ns
    per_w = FULL_BLOCKS // nw
    extra = FULL_BLOCKS - per_w * nw
    max_b = per_w + 1
    mesh = plsc.VectorSubcoreMesh(core_axis_name="c", subcore_axis_name="s")

    @functools.partial(
        pl.kernel,
        mesh=mesh,
        compiler_params=pltpu.CompilerParams(
            use_tc_tiling_on_sc=True, needs_layout_passes=False),
        out_type=jax.ShapeDtypeStruct((V * D,), jnp.float32),
        scratch_types=(
            [pltpu.VMEM((D, 128), jnp.float32) for _ in range(2)]
            + [pltpu.VMEM((128 * D,), jnp.float32) for _ in range(2)]
            + [pltpu.SemaphoreType.DMA for _ in range(4)]
        ),
    )
    def retile(t2, tail, scr, tin0, tin1, rv0, rv1, r0s, r1s, w0s, w1s):
        tins = (tin0, tin1)
        rvs = (rv0, rv1)
        rsem = (r0s, r1s)
        wsem = (w0s, w1s)
        wid = lax.axis_index("s") * nc + lax.axis_index("c")
        n_b = per_w + jnp.where(wid < extra, 1, 0)
        base = wid * per_w + jnp.minimum(wid, extra)
        iota16 = lax.iota(jnp.int32, LANES)
        perm = [(iota16 + k) & 15 for k in range(LANES)]
        ivst = [iota16 * D + perm[k] for k in range(LANES)]

        def start_read(i, b):
            pltpu.async_copy(
                t2.at[:, pl.ds((base + i) * 128, 128)], tins[b], rsem[b])

        def wait_read(b):
            pltpu.make_async_copy(
                t2.at[:, pl.ds(0, 128)], tins[b], rsem[b]).wait()

        def start_write(i, b):
            pltpu.async_copy(
                rvs[b], scr.at[pl.ds((base + i) * 128 * D, 128 * D)], wsem[b])

        def wait_write(b):
            pltpu.make_async_copy(
                rvs[b], scr.at[pl.ds(0, 128 * D)], wsem[b]).wait()

        def transpose(tin_b, rv_b):
            def blk_body(t, _):
                d0 = (t // 8) * LANES
                c0 = (t % 8) * LANES
                cols = iota16 + c0
                off = c0 * D + d0
                vs = [plsc.load_gather(tin_b, [perm[k] + d0, cols])
                      for k in range(LANES)]
                for k in range(LANES):
                    plsc.store_scatter(rv_b, [ivst[k] + off], vs[k])
                return 0

            lax.fori_loop(0, (D // LANES) * 8, blk_body, 0, unroll=False)

        start_read(0, 0)

        def outer(o, _):
            for b in range(2):
                i = o * 2 + b

                @pl.when(i < n_b)
                def _():
                    @pl.when(i + 1 < n_b)
                    def _():
                        start_read(i + 1, 1 - b)

                    wait_read(b)

                    @pl.when(i >= 2)
                    def _():
                        wait_write(b)

                    transpose(tins[b], rvs[b])
                    start_write(i, b)
            return 0

        lax.fori_loop(0, (max_b + 1) // 2, outer, 0, unroll=False)
        wait_write(0)
        wait_write(1)

        @pl.when(wid == nw - 1)
        def _():
            pltpu.sync_copy(tail, tins[0])
            transpose(tins[0], rvs[0])
            pltpu.sync_copy(
                rvs[0].at[pl.ds(0, TAIL_ROWS * D)],
                scr.at[pl.ds(TAIL_START * D, TAIL_ROWS * D)])

    return retile


@functools.lru_cache(maxsize=None)
def _phase2():
    nc, ns = _mesh_info()
    nw = nc * ns
    ib_per_w = NIB // nw
    mesh = plsc.VectorSubcoreMesh(core_axis_name="c", subcore_axis_name="s")

    @functools.partial(
        pl.kernel,
        mesh=mesh,
        compiler_params=pltpu.CompilerParams(
            use_tc_tiling_on_sc=False, needs_layout_passes=False),
        out_type=jax.ShapeDtypeStruct((NJ, D // 8, NIB, 8, 128), jnp.float32),
        scratch_types=(
            [pltpu.VMEM((NJ, ib_per_w, 128), jnp.int32)]
            + [pltpu.VMEM((128, D), jnp.float32) for _ in range(4)]
            + [pltpu.VMEM((D // 8, ib_per_w, 8, 128), jnp.float32)
               for _ in range(2)]
            + [pltpu.SemaphoreType.DMA for _ in range(6)]
        ),
    )
    def emb(xt, scr2, out5, idx_v, r0, r1, r2, r3, tv0, tv1,
            g0, g1, g2, g3, s0, s1):
        rows = (r0, r1, r2, r3)
        gsem = (g0, g1, g2, g3)
        tvs = (tv0, tv1)
        ssem = (s0, s1)
        wid = lax.axis_index("s") * nc + lax.axis_index("c")
        w4 = wid * ib_per_w
        iota16 = lax.iota(jnp.int32, LANES)
        perm = [(iota16 + k) & 15 for k in range(LANES)]
        permhi = [perm[k] >> 3 for k in range(LANES)]
        permlo = [perm[k] & 7 for k in range(LANES)]
        ibl_vec = [jnp.full((LANES,), ibl, jnp.int32)
                   for ibl in range(ib_per_w)]

        pltpu.sync_copy(xt.at[:, pl.ds(w4, ib_per_w)], idx_v)

        def start_gather(q, b):
            pltpu.async_copy(
                scr2.at[idx_v.at[q // ib_per_w, q % ib_per_w]],
                rows[b], gsem[b])

        def wait_gather(b):
            pltpu.make_async_copy(
                scr2.at[idx_v.at[0, 0]], rows[b], gsem[b]).wait()

        def store_slab(j, par, tr):
            return pltpu.async_copy(
                tvs[par].at[tr], out5.at[j, tr, pl.ds(w4, ib_per_w)],
                ssem[par])

        def wait_store(par):
            for tr in range(D // 8):
                pltpu.make_async_copy(
                    tvs[par].at[tr], out5.at[0, tr, pl.ds(w4, ib_per_w)],
                    ssem[par]).wait()

        n_chunks = NJ * ib_per_w
        start_gather(0, 0)
        start_gather(1, 1)

        def outer(jo, _):
            for jj in range(2):
                j = jo * 2 + jj
                par = jj

                @pl.when(j >= 2)
                def _():
                    wait_store(par)

                for ibl in range(ib_per_w):
                    q = j * ib_per_w + ibl

                    @pl.when(q + 2 < n_chunks)
                    def _():
                        start_gather(q + 2, (ibl + 2) % 4)

                    wait_gather(ibl)
                    rb = rows[ibl]
                    tvb = tvs[par]
                    iblv = ibl_vec[ibl]

                    def blk_body(t, _):
                        d0 = (t // 8) * LANES
                        c0 = (t % 8) * LANES
                        trb = (t // 8) * 2
                        cols = iota16 + c0
                        vs = [plsc.load_gather(rb, [cols, perm[k] + d0])
                              for k in range(LANES)]
                        for k in range(LANES):
                            plsc.store_scatter(
                                tvb,
                                [permhi[k] + trb, iblv, permlo[k], cols],
                                vs[k] * SCALE)
                        return 0

                    lax.fori_loop(0, (D // LANES) * 8, blk_body, 0,
                                  unroll=False)

                for tr in range(D // 8):
                    store_slab(j, par, tr)
            return 0

        lax.fori_loop(0, NJ // 2, outer, 0, unroll=False)
        wait_store(0)
        wait_store(1)

    return emb


def kernel(x, table):
    xt = x.T.reshape(NJ, NIB, 128).astype(jnp.int32)
    t2 = table.T
    tail = jnp.pad(table[TAIL_START:].T, ((0, 0), (0, 128 - TAIL_ROWS)))
    scr = _phase1()(t2, tail)
    out5 = _phase2()(xt, scr.reshape(V, D))
    return out5.transpose(2, 4, 0, 1, 3).reshape(NTOK, NJ, D)
